# Initial kernel scaffold; baseline (speedup 1.0000x reference)
import jax
import jax.numpy as jnp
from jax.experimental import pallas as pl


def kernel(labels, eye):
    raise NotImplementedError("write your pallas kernel here")


# speedup vs baseline: 1.0814x; 1.0814x over previous
import functools

import jax
import jax.numpy as jnp
from jax import lax
from jax.experimental import pallas as pl
from jax.experimental.pallas import tpu as pltpu
from jax.experimental.pallas import tpu_sc as plsc

_DIM = 1000
_BATCH = 16384
_LANES = 16
_NW = 32
_ROWS_PER_W = _BATCH // _NW
_CHUNK_ROWS = 64
_CHUNK_WORDS = _CHUNK_ROWS * _DIM
_NCHUNKS = _ROWS_PER_W // _CHUNK_ROWS


def _one_hot_flat(labels):
    mesh = plsc.VectorSubcoreMesh(core_axis_name="c", subcore_axis_name="s")

    @functools.partial(
        pl.kernel,
        mesh=mesh,
        out_type=jax.ShapeDtypeStruct((_BATCH * _DIM,), jnp.float32),
        scratch_types=[
            pltpu.VMEM((_ROWS_PER_W,), jnp.int32),
            pltpu.VMEM((_CHUNK_WORDS,), jnp.float32),
        ],
        compiler_params=pltpu.CompilerParams(needs_layout_passes=False),
    )
    def k(labels_hbm, out_hbm, lab_v, buf_v):
        wid = lax.axis_index("c") * (_NW // 2) + lax.axis_index("s")
        row0 = wid * _ROWS_PER_W
        pltpu.sync_copy(labels_hbm.at[pl.ds(row0 * 1, _ROWS_PER_W)], lab_v)

        zeros16 = jnp.zeros((_LANES,), jnp.float32)
        ones16 = jnp.ones((_LANES,), jnp.float32)
        iota16 = lax.iota(jnp.int32, _LANES)
        row_step = iota16 * _DIM

        def zero_body(i, _):
            buf_v[pl.ds(i * _LANES, _LANES)] = zeros16
            return 0

        lax.fori_loop(0, _CHUNK_WORDS // _LANES, zero_body, 0, unroll=4)

        for c in range(_NCHUNKS):
            for g in range(_CHUNK_ROWS // _LANES):
                lab = lab_v[pl.ds(c * _CHUNK_ROWS + g * _LANES, _LANES)]
                flat_idx = row_step + (g * _LANES * _DIM) + lab
                plsc.store_scatter(buf_v, [flat_idx], ones16)
            out_off = row0 * _DIM + c * _CHUNK_WORDS
            pltpu.sync_copy(buf_v, out_hbm.at[pl.ds(out_off, _CHUNK_WORDS)])
            for g in range(_CHUNK_ROWS // _LANES):
                lab = lab_v[pl.ds(c * _CHUNK_ROWS + g * _LANES, _LANES)]
                flat_idx = row_step + (g * _LANES * _DIM) + lab
                plsc.store_scatter(buf_v, [flat_idx], zeros16)

    return k(labels)


def kernel(labels, eye):
    del eye
    flat = _one_hot_flat(labels.astype(jnp.int32))
    return flat.reshape(_BATCH, _DIM)
